# baseline (speedup 1.0000x reference)
import functools

import jax
import jax.numpy as jnp
from jax import lax
from jax.experimental import pallas as pl
from jax.experimental.pallas import tpu as pltpu
from jax.experimental.pallas import tpu_sc as plsc

NUM_USERS = 25000
NUM_ITEMS = 25000
N = NUM_USERS + NUM_ITEMS
E = 800000
D = 64
B = 4096
GCN_LAYER = 3
REG_LAMBDA = 1e-4
SSL_LAMBDA = 1e-6
ALPHA = 1.0
TEMP = 0.1

SUB = 128
EP_TILE = 50176
EPAD = EP_TILE * 16
ROWS_PER_TILE = EP_TILE // SUB
NROWS_TOTAL = EPAD // SUB
CH_SUB = 14
NCHUNK = ROWS_PER_TILE // CH_SUB
RING = 8
LOOK = 4
NPAD = 50048
ACC_ROWS = NPAD // 16

_f32 = jnp.float32
_i32 = jnp.int32
_bf16 = jnp.bfloat16


def _spmm_body(x_ref, cols_ref, rows_ref, vals_ref, y_ref,
               cvmA, rvmA, vvmA, cvmB, rvmB, vvmB,
               rb0, rb1, rb2, rb3, rb4, rb5, rb6, rb7, zb, acc,
               gsems, ssems, stA, stB):
  c = lax.axis_index("c")
  s = lax.axis_index("s")
  ring = (rb0, rb1, rb2, rb3, rb4, rb5, rb6, rb7)

  def _zero_zb(i, carry):
    zb[i, pl.ds(0, 32)] = jnp.zeros((32,), _bf16)
    return carry
  lax.fori_loop(0, ACC_ROWS, _zero_zb, 0, unroll=8)
  pltpu.sync_copy(zb, acc.at[pl.ds(s * ACC_ROWS, ACC_ROWS)])
  plsc.subcore_barrier()

  def _stage(ch, dstc, dstr, dstv, sem):
    rowbase = s * ROWS_PER_TILE + ch * CH_SUB
    pltpu.async_copy(cols_ref.at[pl.ds(c * NROWS_TOTAL + rowbase, CH_SUB)],
                     dstc, sem)
    pltpu.async_copy(rows_ref.at[pl.ds(rowbase, CH_SUB)], dstr, sem)
    pltpu.async_copy(vals_ref.at[pl.ds(rowbase, CH_SUB)], dstv, sem)

  def _wait_stage(dstc, dstr, dstv, sem):
    pltpu.make_async_copy(cols_ref.at[pl.ds(0, CH_SUB)], dstc, sem).wait()
    pltpu.make_async_copy(rows_ref.at[pl.ds(0, CH_SUB)], dstr, sem).wait()
    pltpu.make_async_copy(vals_ref.at[pl.ds(0, CH_SUB)], dstv, sem).wait()

  def _process(cvm, rvm, vvm, sem):
    _wait_stage(cvm, rvm, vvm, sem)
    gd = {}
    sd = {}
    for j in range(LOOK):
      gd[j] = pltpu.async_copy(x_ref.at[cvm.at[j]], ring[j % RING],
                               gsems.at[j % RING])
    for j in range(CH_SUB):
      buf = ring[j % RING]
      gd[j].wait()

      zeros16 = jnp.zeros((16,), _i32)
      for g in range(SUB // 16):
        vv = vvm[j, pl.ds(g * 16, 16)]
        for k in range(16):
          vb = plsc.bitcast(zeros16 + vv[k], _bf16)
          e = g * 16 + k
          buf[e, pl.ds(0, 32)] = buf[e, pl.ds(0, 32)] * vb

      if j >= 2:
        sd[j - 2].wait()
      if j + LOOK < CH_SUB:
        gd[j + LOOK] = pltpu.async_copy(x_ref.at[cvm.at[j + LOOK]],
                                        ring[(j + LOOK) % RING],
                                        gsems.at[(j + LOOK) % RING])
      sd[j] = pltpu.async_copy(buf, acc.at[rvm.at[j]], ssems.at[j % RING],
                               add=True)
    sd[CH_SUB - 2].wait()
    sd[CH_SUB - 1].wait()

  _stage(0, cvmA, rvmA, vvmA, stA)

  def _pair(t, carry):
    ch0 = 2 * t
    _stage(ch0 + 1, cvmB, rvmB, vvmB, stB)
    _process(cvmA, rvmA, vvmA, stA)
    _stage(jnp.minimum(ch0 + 2, NCHUNK - 1), cvmA, rvmA, vvmA, stA)
    _process(cvmB, rvmB, vvmB, stB)
    return carry
  lax.fori_loop(0, NCHUNK // 2, _pair, 0)
  _wait_stage(cvmA, rvmA, vvmA, stA)

  plsc.subcore_barrier()
  pltpu.sync_copy(acc.at[pl.ds(s * ACC_ROWS, ACC_ROWS)],
                  y_ref.at[pl.ds(c * NPAD + s * ACC_ROWS, ACC_ROWS)])


def _make_spmm():
  mesh = plsc.VectorSubcoreMesh(core_axis_name="c", subcore_axis_name="s")
  stage_set = [
      pltpu.VMEM((CH_SUB, SUB), _i32),
      pltpu.VMEM((CH_SUB, SUB), _i32),
      pltpu.VMEM((CH_SUB, SUB), _i32),
  ]
  return pl.kernel(
      _spmm_body,
      out_type=jax.ShapeDtypeStruct((2 * NPAD, 32), _bf16),
      mesh=mesh,
      scratch_types=stage_set + stage_set + [
          pltpu.VMEM((SUB, 32), _bf16),
          pltpu.VMEM((SUB, 32), _bf16),
          pltpu.VMEM((SUB, 32), _bf16),
          pltpu.VMEM((SUB, 32), _bf16),
          pltpu.VMEM((SUB, 32), _bf16),
          pltpu.VMEM((SUB, 32), _bf16),
          pltpu.VMEM((SUB, 32), _bf16),
          pltpu.VMEM((SUB, 32), _bf16),
          pltpu.VMEM((ACC_ROWS, 32), _bf16),
          pltpu.VMEM_SHARED((NPAD, 32), _bf16),
          pltpu.SemaphoreType.DMA((RING,)),
          pltpu.SemaphoreType.DMA((RING,)),
          pltpu.SemaphoreType.DMA,
          pltpu.SemaphoreType.DMA,
      ],
      compiler_params=pltpu.CompilerParams(use_tc_tiling_on_sc=False,
                                          needs_layout_passes=False),
  )


def _gather2_body(ue_ref, ie_ref, e2_ref, iu_ref, ip_ref,
                  iu2_ref, ip2_ref,
                  g0u_ref, g0p_ref, g2u_ref, g2p_ref,
                  ib, gb64, gb32, sem):
  c = lax.axis_index("c")
  s = lax.axis_index("s")
  wid = s * 2 + c
  r0 = wid * SUB

  for idx_ref, tab_ref, out_ref in ((iu_ref, ue_ref, g0u_ref),
                                    (ip_ref, ie_ref, g0p_ref)):
    pltpu.sync_copy(idx_ref.at[pl.ds(r0, SUB)], ib)
    pltpu.async_copy(tab_ref.at[ib], gb64, sem).wait()
    pltpu.sync_copy(gb64, out_ref.at[pl.ds(r0, SUB)])

  for idx_ref, out_ref in ((iu2_ref, g2u_ref), (ip2_ref, g2p_ref)):
    for t in range(2):
      fr = c * B + s * 256 + t * SUB
      r = s * 256 + t * SUB
      pltpu.sync_copy(idx_ref.at[pl.ds(fr, SUB)], ib)
      pltpu.async_copy(e2_ref.at[ib], gb32, sem).wait()
      pltpu.sync_copy(gb32, out_ref.at[pl.ds(r, SUB), pl.ds(32 * c, 32)])


def _make_gather2():
  mesh = plsc.VectorSubcoreMesh(core_axis_name="c", subcore_axis_name="s")
  o64 = jax.ShapeDtypeStruct((B, D), _f32)
  ob = jax.ShapeDtypeStruct((B, D), _bf16)
  return pl.kernel(
      _gather2_body,
      out_type=(o64, o64, ob, ob),
      mesh=mesh,
      scratch_types=[
          pltpu.VMEM((SUB,), _i32),
          pltpu.VMEM((SUB, D), _f32),
          pltpu.VMEM((SUB, 32), _bf16),
          pltpu.SemaphoreType.DMA,
      ],
      compiler_params=pltpu.CompilerParams(use_tc_tiling_on_sc=False,
                                          needs_layout_passes=False),
  )


def _gatherrest_body(ie_ref, e1_ref, e2_ref, e3_ref,
                     in_ref, iu2_ref, ip2_ref, in2_ref,
                     g0n_ref, g1u_ref, g1p_ref, g1n_ref,
                     g2n_ref, g3u_ref, g3p_ref, g3n_ref,
                     ib, gb64, gb32, sem):
  c = lax.axis_index("c")
  s = lax.axis_index("s")
  wid = s * 2 + c
  r0 = wid * SUB

  pltpu.sync_copy(in_ref.at[pl.ds(r0, SUB)], ib)
  pltpu.async_copy(ie_ref.at[ib], gb64, sem).wait()
  pltpu.sync_copy(gb64, g0n_ref.at[pl.ds(r0, SUB)])

  for idx_ref, tasks in (
      (iu2_ref, ((e1_ref, g1u_ref), (e3_ref, g3u_ref))),
      (ip2_ref, ((e1_ref, g1p_ref), (e3_ref, g3p_ref))),
      (in2_ref, ((e1_ref, g1n_ref), (e2_ref, g2n_ref), (e3_ref, g3n_ref)))):
    for t in range(2):
      fr = c * B + s * 256 + t * SUB
      r = s * 256 + t * SUB
      pltpu.sync_copy(idx_ref.at[pl.ds(fr, SUB)], ib)
      for tab_ref, out_ref in tasks:
        pltpu.async_copy(tab_ref.at[ib], gb32, sem).wait()
        pltpu.sync_copy(gb32, out_ref.at[pl.ds(r, SUB), pl.ds(32 * c, 32)])


def _make_gatherrest():
  mesh = plsc.VectorSubcoreMesh(core_axis_name="c", subcore_axis_name="s")
  o64 = jax.ShapeDtypeStruct((B, D), _f32)
  ob = jax.ShapeDtypeStruct((B, D), _bf16)
  return pl.kernel(
      _gatherrest_body,
      out_type=(o64, ob, ob, ob, ob, ob, ob, ob),
      mesh=mesh,
      scratch_types=[
          pltpu.VMEM((SUB,), _i32),
          pltpu.VMEM((SUB, D), _f32),
          pltpu.VMEM((SUB, 32), _bf16),
          pltpu.SemaphoreType.DMA,
      ],
      compiler_params=pltpu.CompilerParams(use_tc_tiling_on_sc=False,
                                          needs_layout_passes=False),
  )


NP = 25088
TN = 896
NSTEP_HALF = NP // TN
NSTEP = 2 * NSTEP_HALF
NPAD_ROWS = NP - NUM_USERS


_LOG2E = 1.4426950408889634


def _normalize(x):
  sq = x * x
  ones = jnp.ones((x.shape[1], 1), _f32)
  nrm2 = lax.dot_general(sq, ones, (((1,), (0,)), ((), ())),
                         preferred_element_type=_f32)
  return x * lax.rsqrt(nrm2 + 1e-24)


def _ttl_body(epad_ref, g2u_ref, g2p_ref, g0u_ref, g0p_ref,
              tt_ref, qnu_s, qni_s, ttlu_s, ttli_s):
  j = pl.program_id(0)
  phase = j // NSTEP_HALF

  @pl.when(j == 0)
  def _init():
    qnu_s[...] = _normalize(g2u_ref[...].astype(_f32)) * (_LOG2E / TEMP)
    qni_s[...] = _normalize(g2p_ref[...].astype(_f32)) * (_LOG2E / TEMP)
    ttlu_s[...] = jnp.zeros_like(ttlu_s)
    ttli_s[...] = jnp.zeros_like(ttli_s)

  tile = _normalize(epad_ref[0])
  q = jnp.where(phase == 0, qnu_s[...], qni_s[...])
  prod = lax.dot_general(q.astype(jnp.bfloat16), tile.astype(jnp.bfloat16),
                         (((1,), (1,)), ((), ())),
                         preferred_element_type=_f32)
  ex = jnp.exp2(prod).astype(_bf16)
  ones_tn = jnp.ones((TN, 1), _bf16)
  rs = lax.dot_general(ex, ones_tn, (((1,), (0,)), ((), ())),
                       preferred_element_type=_f32)

  @pl.when(phase == 0)
  def _acc_u():
    ttlu_s[:, 0:1] = ttlu_s[:, 0:1] + rs

  @pl.when(phase == 1)
  def _acc_i():
    ttli_s[:, 0:1] = ttli_s[:, 0:1] + rs

  @pl.when(j == NSTEP - 1)
  def _emit():
    pos_u = jnp.exp2(jnp.sum(qnu_s[...] * _normalize(g0u_ref[...]),
                             axis=-1, keepdims=True))
    pos_i = jnp.exp2(jnp.sum(qni_s[...] * _normalize(g0p_ref[...]),
                             axis=-1, keepdims=True))
    lane = lax.broadcasted_iota(_i32, (B, 128), 1)
    vals = jnp.where(lane == 0, ttlu_s[:, 0:1] - float(NPAD_ROWS), 0.0)
    vals = vals + jnp.where(lane == 1, ttli_s[:, 0:1] - float(NPAD_ROWS),
                            0.0)
    vals = vals + jnp.where(lane == 2, pos_u, 0.0)
    vals = vals + jnp.where(lane == 3, pos_i, 0.0)
    tt_ref[...] = vals


def _make_ttl():
  full = pl.BlockSpec((B, D), lambda j: (0, 0))
  return pl.pallas_call(
      _ttl_body,
      grid=(NSTEP,),
      in_specs=[pl.BlockSpec((1, TN, D),
                             lambda j: (j // NSTEP_HALF, j % NSTEP_HALF, 0))]
      + [full] * 4,
      out_specs=pl.BlockSpec((B, 128), lambda j: (0, 0)),
      out_shape=jax.ShapeDtypeStruct((B, 128), _f32),
      scratch_shapes=[
          pltpu.VMEM((B, D), _f32),
          pltpu.VMEM((B, D), _f32),
          pltpu.VMEM((B, 128), _f32),
          pltpu.VMEM((B, 128), _f32),
      ],
      compiler_params=pltpu.CompilerParams(
          dimension_semantics=("arbitrary",)),
  )


def _fin_body(tt_ref,
              g0u_ref, g0p_ref, g0n_ref,
              g1u_ref, g1p_ref, g1n_ref,
              g2u_ref, g2p_ref, g2n_ref,
              g3u_ref, g3p_ref, g3n_ref,
              out_ref):
  g0u = g0u_ref[...]
  g0p = g0p_ref[...]
  g0n = g0n_ref[...]
  u = 0.25 * (g0u + (g1u_ref[...] + g2u_ref[...]
                     + g3u_ref[...]).astype(_f32))
  p = 0.25 * (g0p + (g1p_ref[...] + g2p_ref[...]
                     + g3p_ref[...]).astype(_f32))
  nn = 0.25 * (g0n + (g1n_ref[...] + g2n_ref[...]
                      + g3n_ref[...]).astype(_f32))
  pos_scores = jnp.sum(u * p, axis=-1)
  neg_scores = jnp.sum(u * nn, axis=-1)
  bpr = jnp.mean(jax.nn.softplus(neg_scores - pos_scores))
  reg = REG_LAMBDA * 0.5 * (jnp.sum(g0u * g0u) + jnp.sum(g0p * g0p)
                            + jnp.sum(g0n * g0n)) / B
  ttl_u = tt_ref[:, 0:1]
  ttl_i = tt_ref[:, 1:2]
  pos_u = tt_ref[:, 2:3]
  pos_i = tt_ref[:, 3:4]
  ssl_u = jnp.sum(-jnp.log(pos_u / ttl_u + 1e-7))
  ssl_i = jnp.sum(-jnp.log(pos_i / ttl_i + 1e-7))
  ssl = SSL_LAMBDA * (ssl_u + ALPHA * ssl_i)
  row = lax.broadcasted_iota(_i32, (8, 128), 0)
  lane = lax.broadcasted_iota(_i32, (8, 128), 1)
  vals = jnp.where((row == 0) & (lane == 0), bpr, 0.0)
  vals = vals + jnp.where((row == 0) & (lane == 1), reg, 0.0)
  vals = vals + jnp.where((row == 0) & (lane == 2), ssl, 0.0)
  out_ref[...] = vals


def _make_fin():
  return pl.pallas_call(
      _fin_body,
      out_shape=jax.ShapeDtypeStruct((8, 128), _f32),
  )


def kernel(user, positive, negative, epoch, user_emb, item_emb,
           graph_rows, graph_cols, graph_vals):
  del epoch
  user = user.astype(_i32)
  positive = positive.astype(_i32)
  negative = negative.astype(_i32)

  e0 = jnp.concatenate([user_emb, item_emb], axis=0).astype(_bf16)
  zrows = jnp.zeros((NPAD - N, 32), _bf16)
  x0 = jnp.concatenate([e0[:, :32], zrows, e0[:, 32:], zrows], axis=0)

  pad = EPAD - E
  cols_p = jnp.concatenate([graph_cols.astype(_i32), jnp.zeros(pad, _i32)])
  rows_p = jnp.concatenate([graph_rows.astype(_i32), jnp.zeros(pad, _i32)])
  vals_p = jnp.concatenate([graph_vals, jnp.zeros(pad, _f32)])
  vbits = lax.bitcast_convert_type(vals_p.astype(_bf16),
                                   jnp.uint16).astype(jnp.uint32)
  vdup = lax.bitcast_convert_type(vbits | (vbits << 16), _i32)
  cols_f = jnp.concatenate([cols_p, cols_p + NPAD])
  cols2d = cols_f.reshape(2 * NROWS_TOTAL, SUB)
  rows2d = rows_p.reshape(NROWS_TOTAL, SUB)
  vals2d = vdup.reshape(NROWS_TOTAL, SUB)

  spmm = _make_spmm()
  e1 = spmm(x0, cols2d, rows2d, vals2d)
  e2 = spmm(e1, cols2d, rows2d, vals2d)

  iu2 = jnp.concatenate([user, user + NPAD])
  ip2 = jnp.concatenate([NUM_USERS + positive, NUM_USERS + positive + NPAD])
  in2 = jnp.concatenate([NUM_USERS + negative, NUM_USERS + negative + NPAD])

  g0u, g0p, g2u, g2p = _make_gather2()(user_emb, item_emb, e2,
                                       user, positive, iu2, ip2)

  upad = jnp.zeros((NP - NUM_USERS, D), _f32)
  epad = jnp.stack([jnp.concatenate([user_emb, upad], axis=0),
                    jnp.concatenate([item_emb, upad], axis=0)])
  tt = _make_ttl()(epad, g2u, g2p, g0u, g0p)

  e3 = spmm(e2, cols2d, rows2d, vals2d)
  (g0n, g1u, g1p, g1n, g2n, g3u, g3p, g3n) = _make_gatherrest()(
      item_emb, e1, e2, e3, negative, iu2, ip2, in2)

  out = _make_fin()(tt, g0u, g0p, g0n, g1u, g1p, g1n, g2u, g2p, g2n,
                    g3u, g3p, g3n)
  return out[0, :3]


# speedup vs baseline: 1.1046x; 1.1046x over previous
import functools

import jax
import jax.numpy as jnp
from jax import lax
from jax.experimental import pallas as pl
from jax.experimental.pallas import tpu as pltpu
from jax.experimental.pallas import tpu_sc as plsc

NUM_USERS = 25000
NUM_ITEMS = 25000
N = NUM_USERS + NUM_ITEMS
E = 800000
D = 64
B = 4096
GCN_LAYER = 3
REG_LAMBDA = 1e-4
SSL_LAMBDA = 1e-6
ALPHA = 1.0
TEMP = 0.1

SUB = 128
EP_TILE = 50176
EPAD = EP_TILE * 16
ROWS_PER_TILE = EP_TILE // SUB
NROWS_TOTAL = EPAD // SUB
CH_SUB = 14
NCHUNK = ROWS_PER_TILE // CH_SUB
RING = 8
LOOK = 4
NPAD = 50048
ACC_ROWS = NPAD // 16

_f32 = jnp.float32
_i32 = jnp.int32
_bf16 = jnp.bfloat16


def _spmm_body(x_ref, cols_ref, rows_ref, vals_ref, y_ref,
               cvmA, rvmA, vvmA, cvmB, rvmB, vvmB,
               rb0, rb1, rb2, rb3, rb4, rb5, rb6, rb7, zb, acc,
               gsems, ssems, stA, stB):
  c = lax.axis_index("c")
  s = lax.axis_index("s")
  ring = (rb0, rb1, rb2, rb3, rb4, rb5, rb6, rb7)

  def _zero_zb(i, carry):
    zb[i, pl.ds(0, 32)] = jnp.zeros((32,), _bf16)
    return carry
  lax.fori_loop(0, ACC_ROWS, _zero_zb, 0, unroll=8)
  pltpu.sync_copy(zb, acc.at[pl.ds(s * ACC_ROWS, ACC_ROWS)])
  plsc.subcore_barrier()

  def _stage(ch, dstc, dstr, dstv, sem):
    rowbase = s * ROWS_PER_TILE + ch * CH_SUB
    pltpu.async_copy(cols_ref.at[pl.ds(c * NROWS_TOTAL + rowbase, CH_SUB)],
                     dstc, sem)
    pltpu.async_copy(rows_ref.at[pl.ds(rowbase, CH_SUB)], dstr, sem)
    pltpu.async_copy(vals_ref.at[pl.ds(rowbase, CH_SUB)], dstv, sem)

  def _wait_stage(dstc, dstr, dstv, sem):
    pltpu.make_async_copy(cols_ref.at[pl.ds(0, CH_SUB)], dstc, sem).wait()
    pltpu.make_async_copy(rows_ref.at[pl.ds(0, CH_SUB)], dstr, sem).wait()
    pltpu.make_async_copy(vals_ref.at[pl.ds(0, CH_SUB)], dstv, sem).wait()

  def _process(cvm, rvm, vvm, sem):
    _wait_stage(cvm, rvm, vvm, sem)
    gd = {}
    sd = {}
    for j in range(LOOK):
      gd[j] = pltpu.async_copy(x_ref.at[cvm.at[j]], ring[j % RING],
                               gsems.at[j % RING])
    for j in range(CH_SUB):
      buf = ring[j % RING]
      gd[j].wait()

      zeros16 = jnp.zeros((16,), _i32)
      for g in range(SUB // 16):
        vv = vvm[j, pl.ds(g * 16, 16)]
        for k in range(16):
          vb = plsc.bitcast(zeros16 + vv[k], _bf16)
          e = g * 16 + k
          buf[e, pl.ds(0, 32)] = buf[e, pl.ds(0, 32)] * vb

      if j >= 2:
        sd[j - 2].wait()
      if j + LOOK < CH_SUB:
        gd[j + LOOK] = pltpu.async_copy(x_ref.at[cvm.at[j + LOOK]],
                                        ring[(j + LOOK) % RING],
                                        gsems.at[(j + LOOK) % RING])
      sd[j] = pltpu.async_copy(buf, acc.at[rvm.at[j]], ssems.at[j % RING],
                               add=True)
    sd[CH_SUB - 2].wait()
    sd[CH_SUB - 1].wait()

  _stage(0, cvmA, rvmA, vvmA, stA)

  def _pair(t, carry):
    ch0 = 2 * t
    _stage(ch0 + 1, cvmB, rvmB, vvmB, stB)
    _process(cvmA, rvmA, vvmA, stA)
    _stage(jnp.minimum(ch0 + 2, NCHUNK - 1), cvmA, rvmA, vvmA, stA)
    _process(cvmB, rvmB, vvmB, stB)
    return carry
  lax.fori_loop(0, NCHUNK // 2, _pair, 0)
  _wait_stage(cvmA, rvmA, vvmA, stA)

  plsc.subcore_barrier()
  pltpu.sync_copy(acc.at[pl.ds(s * ACC_ROWS, ACC_ROWS)],
                  y_ref.at[pl.ds(c * NPAD + s * ACC_ROWS, ACC_ROWS)])


def _make_spmm():
  mesh = plsc.VectorSubcoreMesh(core_axis_name="c", subcore_axis_name="s")
  stage_set = [
      pltpu.VMEM((CH_SUB, SUB), _i32),
      pltpu.VMEM((CH_SUB, SUB), _i32),
      pltpu.VMEM((CH_SUB, SUB), _i32),
  ]
  return pl.kernel(
      _spmm_body,
      out_type=jax.ShapeDtypeStruct((2 * NPAD, 32), _bf16),
      mesh=mesh,
      scratch_types=stage_set + stage_set + [
          pltpu.VMEM((SUB, 32), _bf16),
          pltpu.VMEM((SUB, 32), _bf16),
          pltpu.VMEM((SUB, 32), _bf16),
          pltpu.VMEM((SUB, 32), _bf16),
          pltpu.VMEM((SUB, 32), _bf16),
          pltpu.VMEM((SUB, 32), _bf16),
          pltpu.VMEM((SUB, 32), _bf16),
          pltpu.VMEM((SUB, 32), _bf16),
          pltpu.VMEM((ACC_ROWS, 32), _bf16),
          pltpu.VMEM_SHARED((NPAD, 32), _bf16),
          pltpu.SemaphoreType.DMA((RING,)),
          pltpu.SemaphoreType.DMA((RING,)),
          pltpu.SemaphoreType.DMA,
          pltpu.SemaphoreType.DMA,
      ],
      compiler_params=pltpu.CompilerParams(use_tc_tiling_on_sc=False,
                                          needs_layout_passes=False),
  )


def _gather2_body(ue_ref, ie_ref, e2_ref, iu_ref, ip_ref,
                  iu2_ref, ip2_ref,
                  g0u_ref, g0p_ref, g2u_ref, g2p_ref,
                  ib, gb64, gb32, sem):
  c = lax.axis_index("c")
  s = lax.axis_index("s")
  wid = s * 2 + c
  r0 = wid * SUB

  for idx_ref, tab_ref, out_ref in ((iu_ref, ue_ref, g0u_ref),
                                    (ip_ref, ie_ref, g0p_ref)):
    pltpu.sync_copy(idx_ref.at[pl.ds(r0, SUB)], ib)
    pltpu.async_copy(tab_ref.at[ib], gb64, sem).wait()
    pltpu.sync_copy(gb64, out_ref.at[pl.ds(r0, SUB)])

  for idx_ref, out_ref in ((iu2_ref, g2u_ref), (ip2_ref, g2p_ref)):
    for t in range(2):
      fr = c * B + s * 256 + t * SUB
      r = s * 256 + t * SUB
      pltpu.sync_copy(idx_ref.at[pl.ds(fr, SUB)], ib)
      pltpu.async_copy(e2_ref.at[ib], gb32, sem).wait()
      pltpu.sync_copy(gb32, out_ref.at[pl.ds(r, SUB), pl.ds(32 * c, 32)])


def _make_gather2():
  mesh = plsc.VectorSubcoreMesh(core_axis_name="c", subcore_axis_name="s")
  o64 = jax.ShapeDtypeStruct((B, D), _f32)
  ob = jax.ShapeDtypeStruct((B, D), _bf16)
  return pl.kernel(
      _gather2_body,
      out_type=(o64, o64, ob, ob),
      mesh=mesh,
      scratch_types=[
          pltpu.VMEM((SUB,), _i32),
          pltpu.VMEM((SUB, D), _f32),
          pltpu.VMEM((SUB, 32), _bf16),
          pltpu.SemaphoreType.DMA,
      ],
      compiler_params=pltpu.CompilerParams(use_tc_tiling_on_sc=False,
                                          needs_layout_passes=False),
  )


def _gatherrest_body(ie_ref, e1_ref, e2_ref, e3_ref,
                     in_ref, iu2_ref, ip2_ref, in2_ref,
                     g0n_ref, g1u_ref, g1p_ref, g1n_ref,
                     g2n_ref, g3u_ref, g3p_ref, g3n_ref,
                     ib, gb64, gb32, sem):
  c = lax.axis_index("c")
  s = lax.axis_index("s")
  wid = s * 2 + c
  r0 = wid * SUB

  pltpu.sync_copy(in_ref.at[pl.ds(r0, SUB)], ib)
  pltpu.async_copy(ie_ref.at[ib], gb64, sem).wait()
  pltpu.sync_copy(gb64, g0n_ref.at[pl.ds(r0, SUB)])

  for idx_ref, tasks in (
      (iu2_ref, ((e1_ref, g1u_ref), (e3_ref, g3u_ref))),
      (ip2_ref, ((e1_ref, g1p_ref), (e3_ref, g3p_ref))),
      (in2_ref, ((e1_ref, g1n_ref), (e2_ref, g2n_ref), (e3_ref, g3n_ref)))):
    for t in range(2):
      fr = c * B + s * 256 + t * SUB
      r = s * 256 + t * SUB
      pltpu.sync_copy(idx_ref.at[pl.ds(fr, SUB)], ib)
      for tab_ref, out_ref in tasks:
        pltpu.async_copy(tab_ref.at[ib], gb32, sem).wait()
        pltpu.sync_copy(gb32, out_ref.at[pl.ds(r, SUB), pl.ds(32 * c, 32)])


def _make_gatherrest():
  mesh = plsc.VectorSubcoreMesh(core_axis_name="c", subcore_axis_name="s")
  o64 = jax.ShapeDtypeStruct((B, D), _f32)
  ob = jax.ShapeDtypeStruct((B, D), _bf16)
  return pl.kernel(
      _gatherrest_body,
      out_type=(o64, ob, ob, ob, ob, ob, ob, ob),
      mesh=mesh,
      scratch_types=[
          pltpu.VMEM((SUB,), _i32),
          pltpu.VMEM((SUB, D), _f32),
          pltpu.VMEM((SUB, 32), _bf16),
          pltpu.SemaphoreType.DMA,
      ],
      compiler_params=pltpu.CompilerParams(use_tc_tiling_on_sc=False,
                                          needs_layout_passes=False),
  )


NP = 25088
TN = 896
NSTEP_HALF = NP // TN
NSTEP = 2 * NSTEP_HALF
NPAD_ROWS = NP - NUM_USERS


_LOG2E = 1.4426950408889634


def _normalize(x):
  sq = x * x
  ones = jnp.ones((x.shape[1], 1), _f32)
  nrm2 = lax.dot_general(sq, ones, (((1,), (0,)), ((), ())),
                         preferred_element_type=_f32)
  return x * lax.rsqrt(nrm2 + 1e-24)


def _ttl_body(epad_ref, g2u_ref, g2p_ref, g0u_ref, g0p_ref,
              tt_ref, qnu_s, qni_s, ttlu_s, ttli_s):
  j = pl.program_id(0)
  phase = j // NSTEP_HALF

  @pl.when(j == 0)
  def _init():
    qnu_s[...] = _normalize(g2u_ref[...].astype(_f32)) * (_LOG2E / TEMP)
    qni_s[...] = _normalize(g2p_ref[...].astype(_f32)) * (_LOG2E / TEMP)
    ttlu_s[...] = jnp.zeros_like(ttlu_s)
    ttli_s[...] = jnp.zeros_like(ttli_s)

  tile = _normalize(epad_ref[0])
  q = jnp.where(phase == 0, qnu_s[...], qni_s[...])
  prod = lax.dot_general(q.astype(jnp.bfloat16), tile.astype(jnp.bfloat16),
                         (((1,), (1,)), ((), ())),
                         preferred_element_type=_f32)
  rs = jnp.sum(jnp.exp2(prod), axis=1, keepdims=True)

  @pl.when(phase == 0)
  def _acc_u():
    ttlu_s[:, 0:1] = ttlu_s[:, 0:1] + rs

  @pl.when(phase == 1)
  def _acc_i():
    ttli_s[:, 0:1] = ttli_s[:, 0:1] + rs

  @pl.when(j == NSTEP - 1)
  def _emit():
    pos_u = jnp.exp2(jnp.sum(qnu_s[...] * _normalize(g0u_ref[...]),
                             axis=-1, keepdims=True))
    pos_i = jnp.exp2(jnp.sum(qni_s[...] * _normalize(g0p_ref[...]),
                             axis=-1, keepdims=True))
    lane = lax.broadcasted_iota(_i32, (B, 128), 1)
    vals = jnp.where(lane == 0, ttlu_s[:, 0:1] - float(NPAD_ROWS), 0.0)
    vals = vals + jnp.where(lane == 1, ttli_s[:, 0:1] - float(NPAD_ROWS),
                            0.0)
    vals = vals + jnp.where(lane == 2, pos_u, 0.0)
    vals = vals + jnp.where(lane == 3, pos_i, 0.0)
    tt_ref[...] = vals


def _make_ttl():
  full = pl.BlockSpec((B, D), lambda j: (0, 0))
  return pl.pallas_call(
      _ttl_body,
      grid=(NSTEP,),
      in_specs=[pl.BlockSpec((1, TN, D),
                             lambda j: (j // NSTEP_HALF, j % NSTEP_HALF, 0))]
      + [full] * 4,
      out_specs=pl.BlockSpec((B, 128), lambda j: (0, 0)),
      out_shape=jax.ShapeDtypeStruct((B, 128), _f32),
      scratch_shapes=[
          pltpu.VMEM((B, D), _f32),
          pltpu.VMEM((B, D), _f32),
          pltpu.VMEM((B, 128), _f32),
          pltpu.VMEM((B, 128), _f32),
      ],
      compiler_params=pltpu.CompilerParams(
          dimension_semantics=("arbitrary",)),
  )


def _fin_body(tt_ref,
              g0u_ref, g0p_ref, g0n_ref,
              g1u_ref, g1p_ref, g1n_ref,
              g2u_ref, g2p_ref, g2n_ref,
              g3u_ref, g3p_ref, g3n_ref,
              out_ref):
  g0u = g0u_ref[...]
  g0p = g0p_ref[...]
  g0n = g0n_ref[...]
  u = 0.25 * (g0u + (g1u_ref[...] + g2u_ref[...]
                     + g3u_ref[...]).astype(_f32))
  p = 0.25 * (g0p + (g1p_ref[...] + g2p_ref[...]
                     + g3p_ref[...]).astype(_f32))
  nn = 0.25 * (g0n + (g1n_ref[...] + g2n_ref[...]
                      + g3n_ref[...]).astype(_f32))
  pos_scores = jnp.sum(u * p, axis=-1)
  neg_scores = jnp.sum(u * nn, axis=-1)
  bpr = jnp.mean(jax.nn.softplus(neg_scores - pos_scores))
  reg = REG_LAMBDA * 0.5 * (jnp.sum(g0u * g0u) + jnp.sum(g0p * g0p)
                            + jnp.sum(g0n * g0n)) / B
  ttl_u = tt_ref[:, 0:1]
  ttl_i = tt_ref[:, 1:2]
  pos_u = tt_ref[:, 2:3]
  pos_i = tt_ref[:, 3:4]
  ssl_u = jnp.sum(-jnp.log(pos_u / ttl_u + 1e-7))
  ssl_i = jnp.sum(-jnp.log(pos_i / ttl_i + 1e-7))
  ssl = SSL_LAMBDA * (ssl_u + ALPHA * ssl_i)
  row = lax.broadcasted_iota(_i32, (8, 128), 0)
  lane = lax.broadcasted_iota(_i32, (8, 128), 1)
  vals = jnp.where((row == 0) & (lane == 0), bpr, 0.0)
  vals = vals + jnp.where((row == 0) & (lane == 1), reg, 0.0)
  vals = vals + jnp.where((row == 0) & (lane == 2), ssl, 0.0)
  out_ref[...] = vals


def _make_fin():
  return pl.pallas_call(
      _fin_body,
      out_shape=jax.ShapeDtypeStruct((8, 128), _f32),
  )


def kernel(user, positive, negative, epoch, user_emb, item_emb,
           graph_rows, graph_cols, graph_vals):
  del epoch
  user = user.astype(_i32)
  positive = positive.astype(_i32)
  negative = negative.astype(_i32)

  e0 = jnp.concatenate([user_emb, item_emb], axis=0).astype(_bf16)
  zrows = jnp.zeros((NPAD - N, 32), _bf16)
  x0 = jnp.concatenate([e0[:, :32], zrows, e0[:, 32:], zrows], axis=0)

  pad = EPAD - E
  cols_p = jnp.concatenate([graph_cols.astype(_i32), jnp.zeros(pad, _i32)])
  rows_p = jnp.concatenate([graph_rows.astype(_i32), jnp.zeros(pad, _i32)])
  vals_p = jnp.concatenate([graph_vals, jnp.zeros(pad, _f32)])
  vbits = lax.bitcast_convert_type(vals_p.astype(_bf16),
                                   jnp.uint16).astype(jnp.uint32)
  vdup = lax.bitcast_convert_type(vbits | (vbits << 16), _i32)
  cols_f = jnp.concatenate([cols_p, cols_p + NPAD])
  cols2d = cols_f.reshape(2 * NROWS_TOTAL, SUB)
  rows2d = rows_p.reshape(NROWS_TOTAL, SUB)
  vals2d = vdup.reshape(NROWS_TOTAL, SUB)

  spmm = _make_spmm()
  e1 = spmm(x0, cols2d, rows2d, vals2d)
  e2 = spmm(e1, cols2d, rows2d, vals2d)

  iu2 = jnp.concatenate([user, user + NPAD])
  ip2 = jnp.concatenate([NUM_USERS + positive, NUM_USERS + positive + NPAD])
  in2 = jnp.concatenate([NUM_USERS + negative, NUM_USERS + negative + NPAD])

  g0u, g0p, g2u, g2p = _make_gather2()(user_emb, item_emb, e2,
                                       user, positive, iu2, ip2)

  upad = jnp.zeros((NP - NUM_USERS, D), _f32)
  epad = jnp.stack([jnp.concatenate([user_emb, upad], axis=0),
                    jnp.concatenate([item_emb, upad], axis=0)])
  tt = _make_ttl()(epad, g2u, g2p, g0u, g0p)

  e3 = spmm(e2, cols2d, rows2d, vals2d)
  (g0n, g1u, g1p, g1n, g2n, g3u, g3p, g3n) = _make_gatherrest()(
      item_emb, e1, e2, e3, negative, iu2, ip2, in2)

  out = _make_fin()(tt, g0u, g0p, g0n, g1u, g1p, g1n, g2u, g2p, g2n,
                    g3u, g3p, g3n)
  return out[0, :3]


# speedup vs baseline: 1.1372x; 1.0295x over previous
import functools

import jax
import jax.numpy as jnp
from jax import lax
from jax.experimental import pallas as pl
from jax.experimental.pallas import tpu as pltpu
from jax.experimental.pallas import tpu_sc as plsc

NUM_USERS = 25000
NUM_ITEMS = 25000
N = NUM_USERS + NUM_ITEMS
E = 800000
D = 64
B = 4096
GCN_LAYER = 3
REG_LAMBDA = 1e-4
SSL_LAMBDA = 1e-6
ALPHA = 1.0
TEMP = 0.1

SUB = 128
EP_TILE = 50176
EPAD = EP_TILE * 16
ROWS_PER_TILE = EP_TILE // SUB
NROWS_TOTAL = EPAD // SUB
CH_SUB = 14
NCHUNK = ROWS_PER_TILE // CH_SUB
RING = 8
LOOK = 4
NPAD = 50048
ACC_ROWS = NPAD // 16

_f32 = jnp.float32
_i32 = jnp.int32
_bf16 = jnp.bfloat16


def _spmm_body(x_ref, cols_ref, rows_ref, vals_ref, y_ref,
               cvmA, rvmA, vvmA, cvmB, rvmB, vvmB,
               rb0, rb1, rb2, rb3, rb4, rb5, rb6, rb7, zb, acc,
               gsems, ssems, stA, stB):
  c = lax.axis_index("c")
  s = lax.axis_index("s")
  ring = (rb0, rb1, rb2, rb3, rb4, rb5, rb6, rb7)

  def _zero_zb(i, carry):
    zb[i, pl.ds(0, 32)] = jnp.zeros((32,), _bf16)
    return carry
  lax.fori_loop(0, ACC_ROWS, _zero_zb, 0, unroll=8)
  pltpu.sync_copy(zb, acc.at[pl.ds(s * ACC_ROWS, ACC_ROWS)])
  plsc.subcore_barrier()

  def _stage(ch, dstc, dstr, dstv, sem):
    rowbase = s * ROWS_PER_TILE + ch * CH_SUB
    pltpu.async_copy(cols_ref.at[pl.ds(c * NROWS_TOTAL + rowbase, CH_SUB)],
                     dstc, sem)
    pltpu.async_copy(rows_ref.at[pl.ds(rowbase, CH_SUB)], dstr, sem)
    pltpu.async_copy(vals_ref.at[pl.ds(rowbase, CH_SUB)], dstv, sem)

  def _wait_stage(dstc, dstr, dstv, sem):
    pltpu.make_async_copy(cols_ref.at[pl.ds(0, CH_SUB)], dstc, sem).wait()
    pltpu.make_async_copy(rows_ref.at[pl.ds(0, CH_SUB)], dstr, sem).wait()
    pltpu.make_async_copy(vals_ref.at[pl.ds(0, CH_SUB)], dstv, sem).wait()

  def _process(cvm, rvm, vvm, sem):
    _wait_stage(cvm, rvm, vvm, sem)
    gd = {}
    sd = {}
    for j in range(LOOK):
      gd[j] = pltpu.async_copy(x_ref.at[cvm.at[j]], ring[j % RING],
                               gsems.at[j % RING])
    for j in range(CH_SUB):
      buf = ring[j % RING]
      gd[j].wait()

      zeros16 = jnp.zeros((16,), _i32)
      for g in range(SUB // 16):
        vv = vvm[j, pl.ds(g * 16, 16)]
        for k in range(16):
          vb = plsc.bitcast(zeros16 + vv[k], _bf16)
          e = g * 16 + k
          buf[e, pl.ds(0, 32)] = buf[e, pl.ds(0, 32)] * vb

      if j >= 2:
        sd[j - 2].wait()
      if j + LOOK < CH_SUB:
        gd[j + LOOK] = pltpu.async_copy(x_ref.at[cvm.at[j + LOOK]],
                                        ring[(j + LOOK) % RING],
                                        gsems.at[(j + LOOK) % RING])
      sd[j] = pltpu.async_copy(buf, acc.at[rvm.at[j]], ssems.at[j % RING],
                               add=True)
    sd[CH_SUB - 2].wait()
    sd[CH_SUB - 1].wait()

  _stage(0, cvmA, rvmA, vvmA, stA)

  def _pair(t, carry):
    ch0 = 2 * t
    _stage(ch0 + 1, cvmB, rvmB, vvmB, stB)
    _process(cvmA, rvmA, vvmA, stA)
    _stage(jnp.minimum(ch0 + 2, NCHUNK - 1), cvmA, rvmA, vvmA, stA)
    _process(cvmB, rvmB, vvmB, stB)
    return carry
  lax.fori_loop(0, NCHUNK // 2, _pair, 0)
  _wait_stage(cvmA, rvmA, vvmA, stA)

  plsc.subcore_barrier()
  pltpu.sync_copy(acc.at[pl.ds(s * ACC_ROWS, ACC_ROWS)],
                  y_ref.at[pl.ds(c * NPAD + s * ACC_ROWS, ACC_ROWS)])


def _make_spmm():
  mesh = plsc.VectorSubcoreMesh(core_axis_name="c", subcore_axis_name="s")
  stage_set = [
      pltpu.VMEM((CH_SUB, SUB), _i32),
      pltpu.VMEM((CH_SUB, SUB), _i32),
      pltpu.VMEM((CH_SUB, SUB), _i32),
  ]
  return pl.kernel(
      _spmm_body,
      out_type=jax.ShapeDtypeStruct((2 * NPAD, 32), _bf16),
      mesh=mesh,
      scratch_types=stage_set + stage_set + [
          pltpu.VMEM((SUB, 32), _bf16),
          pltpu.VMEM((SUB, 32), _bf16),
          pltpu.VMEM((SUB, 32), _bf16),
          pltpu.VMEM((SUB, 32), _bf16),
          pltpu.VMEM((SUB, 32), _bf16),
          pltpu.VMEM((SUB, 32), _bf16),
          pltpu.VMEM((SUB, 32), _bf16),
          pltpu.VMEM((SUB, 32), _bf16),
          pltpu.VMEM((ACC_ROWS, 32), _bf16),
          pltpu.VMEM_SHARED((NPAD, 32), _bf16),
          pltpu.SemaphoreType.DMA((RING,)),
          pltpu.SemaphoreType.DMA((RING,)),
          pltpu.SemaphoreType.DMA,
          pltpu.SemaphoreType.DMA,
      ],
      compiler_params=pltpu.CompilerParams(use_tc_tiling_on_sc=False,
                                          needs_layout_passes=False),
  )


def _gather2_body(ue_ref, ie_ref, e2_ref, iu_ref, ip_ref,
                  iu2_ref, ip2_ref,
                  g0u_ref, g0p_ref, g2u_ref, g2p_ref,
                  ib, gb64, gb32, sem):
  c = lax.axis_index("c")
  s = lax.axis_index("s")
  wid = s * 2 + c
  r0 = wid * SUB

  for idx_ref, tab_ref, out_ref in ((iu_ref, ue_ref, g0u_ref),
                                    (ip_ref, ie_ref, g0p_ref)):
    pltpu.sync_copy(idx_ref.at[pl.ds(r0, SUB)], ib)
    pltpu.async_copy(tab_ref.at[ib], gb64, sem).wait()
    pltpu.sync_copy(gb64, out_ref.at[pl.ds(r0, SUB)])

  for idx_ref, out_ref in ((iu2_ref, g2u_ref), (ip2_ref, g2p_ref)):
    for t in range(2):
      fr = c * B + s * 256 + t * SUB
      r = s * 256 + t * SUB
      pltpu.sync_copy(idx_ref.at[pl.ds(fr, SUB)], ib)
      pltpu.async_copy(e2_ref.at[ib], gb32, sem).wait()
      pltpu.sync_copy(gb32, out_ref.at[pl.ds(r, SUB), pl.ds(32 * c, 32)])


def _make_gather2():
  mesh = plsc.VectorSubcoreMesh(core_axis_name="c", subcore_axis_name="s")
  o64 = jax.ShapeDtypeStruct((B, D), _f32)
  ob = jax.ShapeDtypeStruct((B, D), _bf16)
  return pl.kernel(
      _gather2_body,
      out_type=(o64, o64, ob, ob),
      mesh=mesh,
      scratch_types=[
          pltpu.VMEM((SUB,), _i32),
          pltpu.VMEM((SUB, D), _f32),
          pltpu.VMEM((SUB, 32), _bf16),
          pltpu.SemaphoreType.DMA,
      ],
      compiler_params=pltpu.CompilerParams(use_tc_tiling_on_sc=False,
                                          needs_layout_passes=False),
  )


def _gatherrest_body(ie_ref, e1_ref, e2_ref, e3_ref,
                     in_ref, iu2_ref, ip2_ref, in2_ref,
                     g0n_ref, g1u_ref, g1p_ref, g1n_ref,
                     g2n_ref, g3u_ref, g3p_ref, g3n_ref,
                     ib, gb64, gb32, sem):
  c = lax.axis_index("c")
  s = lax.axis_index("s")
  wid = s * 2 + c
  r0 = wid * SUB

  pltpu.sync_copy(in_ref.at[pl.ds(r0, SUB)], ib)
  pltpu.async_copy(ie_ref.at[ib], gb64, sem).wait()
  pltpu.sync_copy(gb64, g0n_ref.at[pl.ds(r0, SUB)])

  for idx_ref, tasks in (
      (iu2_ref, ((e1_ref, g1u_ref), (e3_ref, g3u_ref))),
      (ip2_ref, ((e1_ref, g1p_ref), (e3_ref, g3p_ref))),
      (in2_ref, ((e1_ref, g1n_ref), (e2_ref, g2n_ref), (e3_ref, g3n_ref)))):
    for t in range(2):
      fr = c * B + s * 256 + t * SUB
      r = s * 256 + t * SUB
      pltpu.sync_copy(idx_ref.at[pl.ds(fr, SUB)], ib)
      for tab_ref, out_ref in tasks:
        pltpu.async_copy(tab_ref.at[ib], gb32, sem).wait()
        pltpu.sync_copy(gb32, out_ref.at[pl.ds(r, SUB), pl.ds(32 * c, 32)])


def _make_gatherrest():
  mesh = plsc.VectorSubcoreMesh(core_axis_name="c", subcore_axis_name="s")
  o64 = jax.ShapeDtypeStruct((B, D), _f32)
  ob = jax.ShapeDtypeStruct((B, D), _bf16)
  return pl.kernel(
      _gatherrest_body,
      out_type=(o64, ob, ob, ob, ob, ob, ob, ob),
      mesh=mesh,
      scratch_types=[
          pltpu.VMEM((SUB,), _i32),
          pltpu.VMEM((SUB, D), _f32),
          pltpu.VMEM((SUB, 32), _bf16),
          pltpu.SemaphoreType.DMA,
      ],
      compiler_params=pltpu.CompilerParams(use_tc_tiling_on_sc=False,
                                          needs_layout_passes=False),
  )


NP = 25088
TN = 1792
NSTEP_HALF = NP // TN
NSTEP = 2 * NSTEP_HALF
NPAD_ROWS = NP - NUM_USERS


_LOG2E = 1.4426950408889634


def _normalize(x):
  sq = x * x
  ones = jnp.ones((x.shape[1], 1), _f32)
  nrm2 = lax.dot_general(sq, ones, (((1,), (0,)), ((), ())),
                         preferred_element_type=_f32)
  return x * lax.rsqrt(nrm2 + 1e-24)


def _ttl_body(epad_ref, g2u_ref, g2p_ref, g0u_ref, g0p_ref,
              tt_ref, qnu_s, qni_s, ttlu_s, ttli_s):
  j = pl.program_id(0)
  phase = j // NSTEP_HALF

  @pl.when(j == 0)
  def _init():
    qnu_s[...] = _normalize(g2u_ref[...].astype(_f32)) * (_LOG2E / TEMP)
    qni_s[...] = _normalize(g2p_ref[...].astype(_f32)) * (_LOG2E / TEMP)
    ttlu_s[...] = jnp.zeros_like(ttlu_s)
    ttli_s[...] = jnp.zeros_like(ttli_s)

  tile = _normalize(epad_ref[0])
  q = jnp.where(phase == 0, qnu_s[...], qni_s[...])
  prod = lax.dot_general(q.astype(jnp.bfloat16), tile.astype(jnp.bfloat16),
                         (((1,), (1,)), ((), ())),
                         preferred_element_type=_f32)
  rs = jnp.sum(jnp.exp2(prod), axis=1, keepdims=True)

  @pl.when(phase == 0)
  def _acc_u():
    ttlu_s[:, 0:1] = ttlu_s[:, 0:1] + rs

  @pl.when(phase == 1)
  def _acc_i():
    ttli_s[:, 0:1] = ttli_s[:, 0:1] + rs

  @pl.when(j == NSTEP - 1)
  def _emit():
    pos_u = jnp.exp2(jnp.sum(qnu_s[...] * _normalize(g0u_ref[...]),
                             axis=-1, keepdims=True))
    pos_i = jnp.exp2(jnp.sum(qni_s[...] * _normalize(g0p_ref[...]),
                             axis=-1, keepdims=True))
    lane = lax.broadcasted_iota(_i32, (B, 128), 1)
    vals = jnp.where(lane == 0, ttlu_s[:, 0:1] - float(NPAD_ROWS), 0.0)
    vals = vals + jnp.where(lane == 1, ttli_s[:, 0:1] - float(NPAD_ROWS),
                            0.0)
    vals = vals + jnp.where(lane == 2, pos_u, 0.0)
    vals = vals + jnp.where(lane == 3, pos_i, 0.0)
    tt_ref[...] = vals


def _make_ttl():
  full = pl.BlockSpec((B, D), lambda j: (0, 0))
  return pl.pallas_call(
      _ttl_body,
      grid=(NSTEP,),
      in_specs=[pl.BlockSpec((1, TN, D),
                             lambda j: (j // NSTEP_HALF, j % NSTEP_HALF, 0))]
      + [full] * 4,
      out_specs=pl.BlockSpec((B, 128), lambda j: (0, 0)),
      out_shape=jax.ShapeDtypeStruct((B, 128), _f32),
      scratch_shapes=[
          pltpu.VMEM((B, D), _f32),
          pltpu.VMEM((B, D), _f32),
          pltpu.VMEM((B, 128), _f32),
          pltpu.VMEM((B, 128), _f32),
      ],
      compiler_params=pltpu.CompilerParams(
          dimension_semantics=("arbitrary",)),
  )


def _fin_body(tt_ref,
              g0u_ref, g0p_ref, g0n_ref,
              g1u_ref, g1p_ref, g1n_ref,
              g2u_ref, g2p_ref, g2n_ref,
              g3u_ref, g3p_ref, g3n_ref,
              out_ref):
  g0u = g0u_ref[...]
  g0p = g0p_ref[...]
  g0n = g0n_ref[...]
  u = 0.25 * (g0u + (g1u_ref[...] + g2u_ref[...]
                     + g3u_ref[...]).astype(_f32))
  p = 0.25 * (g0p + (g1p_ref[...] + g2p_ref[...]
                     + g3p_ref[...]).astype(_f32))
  nn = 0.25 * (g0n + (g1n_ref[...] + g2n_ref[...]
                      + g3n_ref[...]).astype(_f32))
  pos_scores = jnp.sum(u * p, axis=-1)
  neg_scores = jnp.sum(u * nn, axis=-1)
  bpr = jnp.mean(jax.nn.softplus(neg_scores - pos_scores))
  reg = REG_LAMBDA * 0.5 * (jnp.sum(g0u * g0u) + jnp.sum(g0p * g0p)
                            + jnp.sum(g0n * g0n)) / B
  ttl_u = tt_ref[:, 0:1]
  ttl_i = tt_ref[:, 1:2]
  pos_u = tt_ref[:, 2:3]
  pos_i = tt_ref[:, 3:4]
  ssl_u = jnp.sum(-jnp.log(pos_u / ttl_u + 1e-7))
  ssl_i = jnp.sum(-jnp.log(pos_i / ttl_i + 1e-7))
  ssl = SSL_LAMBDA * (ssl_u + ALPHA * ssl_i)
  row = lax.broadcasted_iota(_i32, (8, 128), 0)
  lane = lax.broadcasted_iota(_i32, (8, 128), 1)
  vals = jnp.where((row == 0) & (lane == 0), bpr, 0.0)
  vals = vals + jnp.where((row == 0) & (lane == 1), reg, 0.0)
  vals = vals + jnp.where((row == 0) & (lane == 2), ssl, 0.0)
  out_ref[...] = vals


def _make_fin():
  return pl.pallas_call(
      _fin_body,
      out_shape=jax.ShapeDtypeStruct((8, 128), _f32),
  )


def kernel(user, positive, negative, epoch, user_emb, item_emb,
           graph_rows, graph_cols, graph_vals):
  del epoch
  user = user.astype(_i32)
  positive = positive.astype(_i32)
  negative = negative.astype(_i32)

  e0 = jnp.concatenate([user_emb, item_emb], axis=0).astype(_bf16)
  zrows = jnp.zeros((NPAD - N, 32), _bf16)
  x0 = jnp.concatenate([e0[:, :32], zrows, e0[:, 32:], zrows], axis=0)

  pad = EPAD - E
  cols_p = jnp.concatenate([graph_cols.astype(_i32), jnp.zeros(pad, _i32)])
  rows_p = jnp.concatenate([graph_rows.astype(_i32), jnp.zeros(pad, _i32)])
  vals_p = jnp.concatenate([graph_vals, jnp.zeros(pad, _f32)])
  vbits = lax.bitcast_convert_type(vals_p.astype(_bf16),
                                   jnp.uint16).astype(jnp.uint32)
  vdup = lax.bitcast_convert_type(vbits | (vbits << 16), _i32)
  cols_f = jnp.concatenate([cols_p, cols_p + NPAD])
  cols2d = cols_f.reshape(2 * NROWS_TOTAL, SUB)
  rows2d = rows_p.reshape(NROWS_TOTAL, SUB)
  vals2d = vdup.reshape(NROWS_TOTAL, SUB)

  spmm = _make_spmm()
  e1 = spmm(x0, cols2d, rows2d, vals2d)
  e2 = spmm(e1, cols2d, rows2d, vals2d)

  iu2 = jnp.concatenate([user, user + NPAD])
  ip2 = jnp.concatenate([NUM_USERS + positive, NUM_USERS + positive + NPAD])
  in2 = jnp.concatenate([NUM_USERS + negative, NUM_USERS + negative + NPAD])

  g0u, g0p, g2u, g2p = _make_gather2()(user_emb, item_emb, e2,
                                       user, positive, iu2, ip2)

  upad = jnp.zeros((NP - NUM_USERS, D), _f32)
  epad = jnp.stack([jnp.concatenate([user_emb, upad], axis=0),
                    jnp.concatenate([item_emb, upad], axis=0)])
  tt = _make_ttl()(epad, g2u, g2p, g0u, g0p)

  e3 = spmm(e2, cols2d, rows2d, vals2d)
  (g0n, g1u, g1p, g1n, g2n, g3u, g3p, g3n) = _make_gatherrest()(
      item_emb, e1, e2, e3, negative, iu2, ip2, in2)

  out = _make_fin()(tt, g0u, g0p, g0n, g1u, g1p, g1n, g2u, g2p, g2n,
                    g3u, g3p, g3n)
  return out[0, :3]
